# baseline (speedup 1.0000x reference)
import jax
import jax.numpy as jnp
from jax.experimental import pallas as pl
from jax.experimental.pallas import tpu as pltpu

B, N, C = 8, 16384, 80
M = B * N
BM = 2048


def _body(w_ref, x_ref, z_ref, m_ref, oh_ref, o_ref):
    x = x_ref[...]
    z = z_ref[...]
    m = m_ref[...]
    oh = oh_ref[...]
    w = w_ref[...]

    sw = jnp.sum(oh * w, axis=1, keepdims=True)
    sig = jax.nn.sigmoid(x)
    sp = jnp.maximum(x, 0.0) + jnp.log1p(jnp.exp(-jnp.abs(x)))
    neg = sp * sig * sig
    d = z - sig
    pos = (sp - x * z) * d * d
    o_ref[...] = sw * jnp.where(m, pos, neg)


def kernel(pred_score, gt_score, gt_target_pos_mask, labels_one_hot, weights):
    x2 = pred_score.reshape(M, C)
    z2 = gt_score.reshape(M, C)
    m2 = gt_target_pos_mask.reshape(M, C)
    oh2 = labels_one_hot.reshape(M, C)
    w2 = weights.reshape(1, C)

    grid = (M // BM,)
    row_spec = pl.BlockSpec((BM, C), lambda i: (i, 0))
    out = pl.pallas_call(
        _body,
        grid=grid,
        in_specs=[
            pl.BlockSpec((1, C), lambda i: (0, 0)),
            row_spec,
            row_spec,
            row_spec,
            row_spec,
        ],
        out_specs=row_spec,
        out_shape=jax.ShapeDtypeStruct((M, C), jnp.float32),
    )(w2, x2, z2, m2, oh2)
    return out.reshape(B, N, C)
